# Initial kernel scaffold; baseline (speedup 1.0000x reference)
import jax
import jax.numpy as jnp
from jax.experimental import pallas as pl


def kernel(tile_resource, tile_dicenum, tile_pos, port_resource, port_pos, struct_owner, struct_type, struct_pos, road_owner, road_a, road_b, tiletype_embed, resource_embed, dicenum_embed, position_embed, tile_proj_w, tile_proj_b, port_resource_embed, port_position_embed, port_proj_w, port_proj_b, owner_embed, structure_type_embed, node_pos_embed, struct_proj_w, struct_proj_b, road_proj_w, road_proj_b):
    raise NotImplementedError("write your pallas kernel here")


# speedup vs baseline: 5.4600x; 5.4600x over previous
import functools

import jax
import jax.numpy as jnp
from jax import lax
from jax.experimental import pallas as pl
from jax.experimental.pallas import tpu as pltpu
from jax.experimental.pallas import tpu_sc as plsc

B = 16384
D = 64

_N_TILE, _N_PORT, _N_STRUCT, _N_ROAD = 1368, 54, 432, 11664
_OFF_T, _OFF_P, _OFF_S, _OFF_R = 0, 1368, 1424, 1856
_NT = 13520

_W = (19, 9, 54, 72)
_W8 = (24, 16, 56, 72)
_COL = (0, 19, 28, 82)

_NC, _NS = 2, 16
_NW = _NC * _NS
_BPW = B // _NW
_CB = 8
_NCHUNK = _BPW // _CB


def _table_body(tiletype, resource, dicenum, position, tw, tb,
                port_res, port_pos, pw, pb,
                owner, stype, node, sw, sb, rw, rb, out_ref):
    f32 = jnp.float32

    def proj(e_ref, w_ref, k):
        return lax.dot_general(e_ref[...], w_ref[64 * k:64 * (k + 1), :],
                               (((1,), (0,)), ((), ())),
                               preferred_element_type=f32)

    def oh_mm(vals_col, n, tab):
        npad = vals_col.shape[0]
        cols = lax.broadcasted_iota(jnp.int32, (npad, n), 1)
        return lax.dot_general((cols == vals_col).astype(f32), tab,
                               (((1,), (0,)), ((), ())),
                               preferred_element_type=f32)

    def const_row(ttype_row, w_ref, k, b_ref):
        return lax.dot_general(ttype_row, w_ref[64 * k:64 * (k + 1), :],
                               (((1,), (0,)), ((), ())),
                               preferred_element_type=f32) + b_ref[...]

    tt = tiletype[...]

    r = lax.broadcasted_iota(jnp.int32, (_N_TILE, 1), 0)
    tiles_tab = (oh_mm(r // 228, 6, proj(resource, tw, 0))
                 + oh_mm((r // 19) % 12, 12, proj(dicenum, tw, 1))
                 + oh_mm(r % 19, 19, proj(position, tw, 2))
                 + const_row(tt[0:1, :], tw, 3, tb))
    out_ref[0:_N_TILE, :] = tiles_tab

    r = lax.broadcasted_iota(jnp.int32, (56, 1), 0)
    ports_tab = (oh_mm(r // 9, 6, proj(port_res, pw, 0))
                 + oh_mm(r % 9, 9, proj(port_pos, pw, 1))
                 + const_row(tt[1:2, :], pw, 2, pb))
    out_ref[_OFF_P:_OFF_P + 56, :] = ports_tab

    r = lax.broadcasted_iota(jnp.int32, (_N_STRUCT, 1), 0)
    struct_tab = (oh_mm(r // 108, 4, proj(owner, sw, 0))
                  + oh_mm((r // 54) % 2, 2, proj(stype, sw, 1))
                  + oh_mm(r % 54, 54, proj(node, sw, 2))
                  + const_row(tt[2:3, :], sw, 3, sb))
    out_ref[_OFF_S:_OFF_S + _N_STRUCT, :] = struct_tab

    r = lax.broadcasted_iota(jnp.int32, (_N_ROAD, 1), 0)
    roads_tab = (oh_mm(r // 2916, 4, proj(owner, rw, 0))
                 + oh_mm((r // 54) % 54, 54, proj(node, rw, 1))
                 + oh_mm(r % 54, 54, proj(node, rw, 2))
                 + const_row(tt[3:4, :], rw, 3, rb))
    out_ref[_OFF_R:_OFF_R + _N_ROAD, :] = roads_tab


def _build_table(tiletype_embed, resource_embed, dicenum_embed, position_embed,
                 tile_proj_w, tile_proj_b, port_resource_embed,
                 port_position_embed, port_proj_w, port_proj_b, owner_embed,
                 structure_type_embed, node_pos_embed, struct_proj_w,
                 struct_proj_b, road_proj_w, road_proj_b):
    return pl.pallas_call(
        _table_body,
        out_shape=jax.ShapeDtypeStruct((_NT, D), jnp.float32),
    )(tiletype_embed, resource_embed, dicenum_embed, position_embed,
      tile_proj_w, tile_proj_b.reshape(1, D), port_resource_embed,
      port_position_embed, port_proj_w, port_proj_b.reshape(1, D),
      owner_embed, structure_type_embed, node_pos_embed, struct_proj_w,
      struct_proj_b.reshape(1, D), road_proj_w, road_proj_b.reshape(1, D))


def _cidx_body(tr, td, tp, pr, pp, so, st, sp, ro, ra, rb,
               out_t, out_p, out_s, out_r):
    out_t[...] = (tr[...] * 12 + td[...]) * 19 + tp[...]
    out_p[...] = _OFF_P + pr[...] * 9 + pp[...]
    out_s[...] = _OFF_S + (so[...] * 2 + st[...]) * 54 + sp[...]
    out_r[...] = _OFF_R + (ro[...] * 54 + ra[...]) * 54 + rb[...]


def _build_cidx(tile_resource, tile_dicenum, tile_pos, port_resource, port_pos,
                struct_owner, struct_type, struct_pos, road_owner, road_a,
                road_b):
    blk = 512
    grid = (B // blk,)

    def spec(w):
        return pl.BlockSpec((blk, w), lambda i: (i, 0))

    in_w = (19, 19, 19, 9, 9, 54, 54, 54, 72, 72, 72)
    out_w = (19, 9, 54, 72)
    return pl.pallas_call(
        _cidx_body,
        grid=grid,
        in_specs=[spec(w) for w in in_w],
        out_specs=[spec(w) for w in out_w],
        out_shape=[jax.ShapeDtypeStruct((B, w), jnp.int32) for w in out_w],
    )(tile_resource, tile_dicenum, tile_pos, port_resource, port_pos,
      struct_owner, struct_type, struct_pos, road_owner, road_a, road_b)


def _sc_gather_body(table, ct, cp, cs, cr, out,
                    it, ip, is_, ir, st, sp_, ss, sr, sem):
    wid = lax.axis_index("s") * _NC + lax.axis_index("c")
    base = wid * _BPW
    cidxs = (ct, cp, cs, cr)
    idxbufs = (it, ip, is_, ir)
    stages = (st, sp_, ss, sr)

    def chunk_body(g, carry):
        bb0 = base + g * _CB
        for k in range(4):
            pltpu.sync_copy(cidxs[k].at[pl.ds(bb0 * _W8[k], _CB * _W8[k])],
                            idxbufs[k])
        copies = []
        for k in range(4):
            for b in range(_CB):
                copies.append(pltpu.async_copy(
                    table.at[idxbufs[k].at[pl.ds(b * _W8[k], _W8[k])]],
                    stages[k].at[b], sem))
        for c in copies:
            c.wait()
        for k in range(4):
            pltpu.sync_copy(
                stages[k].at[:, pl.ds(0, _W[k])],
                out.at[pl.ds(bb0, _CB), pl.ds(_COL[k], _W[k])])
        return carry

    lax.fori_loop(0, _NCHUNK, chunk_body, 0)


@functools.cache
def _sc_gather():
    return pl.kernel(
        _sc_gather_body,
        out_type=jax.ShapeDtypeStruct((B, 154, D), jnp.float32),
        mesh=plsc.VectorSubcoreMesh(core_axis_name="c", subcore_axis_name="s",
                                    num_cores=_NC, num_subcores=_NS),
        compiler_params=pltpu.CompilerParams(use_tc_tiling_on_sc=False),
        scratch_types=[
            pltpu.VMEM((_CB * 24,), jnp.int32),
            pltpu.VMEM((_CB * 16,), jnp.int32),
            pltpu.VMEM((_CB * 56,), jnp.int32),
            pltpu.VMEM((_CB * 72,), jnp.int32),
            pltpu.VMEM((_CB, 24, D), jnp.float32),
            pltpu.VMEM((_CB, 16, D), jnp.float32),
            pltpu.VMEM((_CB, 56, D), jnp.float32),
            pltpu.VMEM((_CB, 72, D), jnp.float32),
            pltpu.SemaphoreType.DMA,
        ],
    )


def kernel(tile_resource, tile_dicenum, tile_pos, port_resource, port_pos,
           struct_owner, struct_type, struct_pos, road_owner, road_a, road_b,
           tiletype_embed, resource_embed, dicenum_embed, position_embed,
           tile_proj_w, tile_proj_b, port_resource_embed, port_position_embed,
           port_proj_w, port_proj_b, owner_embed, structure_type_embed,
           node_pos_embed, struct_proj_w, struct_proj_b, road_proj_w,
           road_proj_b):
    table = _build_table(
        tiletype_embed, resource_embed, dicenum_embed, position_embed,
        tile_proj_w, tile_proj_b, port_resource_embed, port_position_embed,
        port_proj_w, port_proj_b, owner_embed, structure_type_embed,
        node_pos_embed, struct_proj_w, struct_proj_b, road_proj_w, road_proj_b)
    ct, cp, cs, cr = _build_cidx(
        tile_resource, tile_dicenum, tile_pos, port_resource, port_pos,
        struct_owner, struct_type, struct_pos, road_owner, road_a, road_b)
    ctp = jnp.pad(ct, ((0, 0), (0, 5))).reshape(-1)
    cpp = jnp.pad(cp, ((0, 0), (0, 7))).reshape(-1)
    csp = jnp.pad(cs, ((0, 0), (0, 2))).reshape(-1)
    crp = cr.reshape(-1)
    return _sc_gather()(table, ctp, cpp, csp, crp)


# speedup vs baseline: 19.3197x; 3.5384x over previous
import functools

import jax
import jax.numpy as jnp
from jax import lax
from jax.experimental import pallas as pl
from jax.experimental.pallas import tpu as pltpu
from jax.experimental.pallas import tpu_sc as plsc

B = 16384
D = 64

_N_TILE, _N_PORT, _N_STRUCT, _N_ROAD = 1368, 54, 432, 11664
_OFF_T, _OFF_P, _OFF_S, _OFF_R = 0, 1368, 1424, 1856
_NT = 13520

_W = (19, 9, 54, 72)
_W8 = (24, 16, 56, 72)
_COL = (0, 19, 28, 82)

_NC, _NS = 2, 16
_NW = _NC * _NS
_BPW = B // _NW
_CB = 8
_NCHUNK = _BPW // _CB


def _table_body(tiletype, resource, dicenum, position, tw, tb,
                port_res, port_pos, pw, pb,
                owner, stype, node, sw, sb, rw, rb, out_ref):
    f32 = jnp.float32

    def proj(e_ref, w_ref, k):
        return lax.dot_general(e_ref[...], w_ref[64 * k:64 * (k + 1), :],
                               (((1,), (0,)), ((), ())),
                               preferred_element_type=f32)

    def oh_mm(vals_col, n, tab):
        npad = vals_col.shape[0]
        cols = lax.broadcasted_iota(jnp.int32, (npad, n), 1)
        return lax.dot_general((cols == vals_col).astype(f32), tab,
                               (((1,), (0,)), ((), ())),
                               preferred_element_type=f32)

    def const_row(ttype_row, w_ref, k, b_ref):
        return lax.dot_general(ttype_row, w_ref[64 * k:64 * (k + 1), :],
                               (((1,), (0,)), ((), ())),
                               preferred_element_type=f32) + b_ref[...]

    tt = tiletype[...]

    r = lax.broadcasted_iota(jnp.int32, (_N_TILE, 1), 0)
    tiles_tab = (oh_mm(r // 228, 6, proj(resource, tw, 0))
                 + oh_mm((r // 19) % 12, 12, proj(dicenum, tw, 1))
                 + oh_mm(r % 19, 19, proj(position, tw, 2))
                 + const_row(tt[0:1, :], tw, 3, tb))
    out_ref[0:_N_TILE, :] = tiles_tab

    r = lax.broadcasted_iota(jnp.int32, (56, 1), 0)
    ports_tab = (oh_mm(r // 9, 6, proj(port_res, pw, 0))
                 + oh_mm(r % 9, 9, proj(port_pos, pw, 1))
                 + const_row(tt[1:2, :], pw, 2, pb))
    out_ref[_OFF_P:_OFF_P + 56, :] = ports_tab

    r = lax.broadcasted_iota(jnp.int32, (_N_STRUCT, 1), 0)
    struct_tab = (oh_mm(r // 108, 4, proj(owner, sw, 0))
                  + oh_mm((r // 54) % 2, 2, proj(stype, sw, 1))
                  + oh_mm(r % 54, 54, proj(node, sw, 2))
                  + const_row(tt[2:3, :], sw, 3, sb))
    out_ref[_OFF_S:_OFF_S + _N_STRUCT, :] = struct_tab

    r = lax.broadcasted_iota(jnp.int32, (_N_ROAD, 1), 0)
    roads_tab = (oh_mm(r // 2916, 4, proj(owner, rw, 0))
                 + oh_mm((r // 54) % 54, 54, proj(node, rw, 1))
                 + oh_mm(r % 54, 54, proj(node, rw, 2))
                 + const_row(tt[3:4, :], rw, 3, rb))
    out_ref[_OFF_R:_OFF_R + _N_ROAD, :] = roads_tab


def _build_table(tiletype_embed, resource_embed, dicenum_embed, position_embed,
                 tile_proj_w, tile_proj_b, port_resource_embed,
                 port_position_embed, port_proj_w, port_proj_b, owner_embed,
                 structure_type_embed, node_pos_embed, struct_proj_w,
                 struct_proj_b, road_proj_w, road_proj_b):
    return pl.pallas_call(
        _table_body,
        out_shape=jax.ShapeDtypeStruct((_NT, D), jnp.float32),
    )(tiletype_embed, resource_embed, dicenum_embed, position_embed,
      tile_proj_w, tile_proj_b.reshape(1, D), port_resource_embed,
      port_position_embed, port_proj_w, port_proj_b.reshape(1, D),
      owner_embed, structure_type_embed, node_pos_embed, struct_proj_w,
      struct_proj_b.reshape(1, D), road_proj_w, road_proj_b.reshape(1, D))


def _cidx_body(tr, td, tp, pr, pp, so, st, sp, ro, ra, rb, out):
    out[:, 0:19] = (tr[...] * 12 + td[...]) * 19 + tp[...]
    out[:, 19:28] = _OFF_P + pr[...] * 9 + pp[...]
    out[:, 28:82] = _OFF_S + (so[...] * 2 + st[...]) * 54 + sp[...]
    out[:, 82:154] = _OFF_R + (ro[...] * 54 + ra[...]) * 54 + rb[...]


def _build_cidx(tile_resource, tile_dicenum, tile_pos, port_resource, port_pos,
                struct_owner, struct_type, struct_pos, road_owner, road_a,
                road_b):
    blk = 512
    grid = (B // blk,)

    def spec(w):
        return pl.BlockSpec((blk, w), lambda i: (i, 0))

    in_w = (19, 19, 19, 9, 9, 54, 54, 54, 72, 72, 72)
    return pl.pallas_call(
        _cidx_body,
        grid=grid,
        in_specs=[spec(w) for w in in_w],
        out_specs=spec(154),
        out_shape=jax.ShapeDtypeStruct((B, 154), jnp.int32),
    )(tile_resource, tile_dicenum, tile_pos, port_resource, port_pos,
      struct_owner, struct_type, struct_pos, road_owner, road_a, road_b)


_RPW = (B // _NW) * 154
_RC = 704
_NCH = _RPW // _RC


def _sc_gather_body(table, cidx, out, shared_tab, idxb, stage, sem):
    cid = lax.axis_index("c")
    sid = lax.axis_index("s")
    wid = sid * _NC + cid

    @pl.when(sid == 0)
    def _():
        pltpu.sync_copy(table, shared_tab)
    plsc.subcore_barrier()

    base = wid * _RPW

    def chunk_body(g, carry):
        r0 = base + g * _RC
        pltpu.sync_copy(cidx.at[pl.ds(r0, _RC)], idxb)
        cps = []
        for p0 in range(0, _RC, 128):
            n = min(128, _RC - p0)
            cps.append(pltpu.async_copy(
                shared_tab.at[idxb.at[pl.ds(p0, n)]],
                stage.at[pl.ds(p0, n)], sem))
        for c in cps:
            c.wait()
        pltpu.sync_copy(stage, out.at[pl.ds(r0, _RC)])
        return carry

    lax.fori_loop(0, _NCH, chunk_body, 0)


@functools.cache
def _sc_gather():
    return pl.kernel(
        _sc_gather_body,
        out_type=jax.ShapeDtypeStruct((B * 154, D), jnp.float32),
        mesh=plsc.VectorSubcoreMesh(core_axis_name="c", subcore_axis_name="s",
                                    num_cores=_NC, num_subcores=_NS),
        compiler_params=pltpu.CompilerParams(use_tc_tiling_on_sc=False),
        scratch_types=[
            pltpu.VMEM_SHARED((_NT, D), jnp.float32),
            pltpu.VMEM((_RC,), jnp.int32),
            pltpu.VMEM((_RC, D), jnp.float32),
            pltpu.SemaphoreType.DMA,
        ],
    )


def kernel(tile_resource, tile_dicenum, tile_pos, port_resource, port_pos,
           struct_owner, struct_type, struct_pos, road_owner, road_a, road_b,
           tiletype_embed, resource_embed, dicenum_embed, position_embed,
           tile_proj_w, tile_proj_b, port_resource_embed, port_position_embed,
           port_proj_w, port_proj_b, owner_embed, structure_type_embed,
           node_pos_embed, struct_proj_w, struct_proj_b, road_proj_w,
           road_proj_b):
    table = _build_table(
        tiletype_embed, resource_embed, dicenum_embed, position_embed,
        tile_proj_w, tile_proj_b, port_resource_embed, port_position_embed,
        port_proj_w, port_proj_b, owner_embed, structure_type_embed,
        node_pos_embed, struct_proj_w, struct_proj_b, road_proj_w, road_proj_b)
    cidx = _build_cidx(
        tile_resource, tile_dicenum, tile_pos, port_resource, port_pos,
        struct_owner, struct_type, struct_pos, road_owner, road_a, road_b)
    out = _sc_gather()(table, cidx.reshape(-1))
    return out.reshape(B, 154, D)
